# baseline (device time: 11039 ns/iter reference)
import jax
import jax.numpy as jnp
from jax import lax
from jax.experimental import pallas as pl
from jax.experimental.pallas import tpu as pltpu

N_X = 2
BLOCK_M = 512


def kernel(x):
    m_per, n_per = x.shape
    m_global = N_X * m_per
    num_blocks = m_per // BLOCK_M

    def body(x_ref, out_ref, acc_ref, comm_ref, send_sem, recv_sem):
        i = pl.program_id(0)
        my_x = lax.axis_index("x")
        my_y = lax.axis_index("y")
        nbr = (1 - my_x, my_y)
        barrier_sem = pltpu.get_barrier_semaphore()

        @pl.when(i == 0)
        def _():
            pl.semaphore_signal(
                barrier_sem, inc=1, device_id=nbr,
                device_id_type=pl.DeviceIdType.MESH,
            )
            pl.semaphore_wait(barrier_sem, 1)

        part = jnp.sum(x_ref[...].astype(jnp.float32), axis=0, keepdims=True)

        @pl.when(i == 0)
        def _():
            acc_ref[...] = part

        @pl.when(i > 0)
        def _():
            acc_ref[...] += part

        @pl.when(i == num_blocks - 1)
        def _():
            rdma = pltpu.make_async_remote_copy(
                src_ref=acc_ref,
                dst_ref=comm_ref,
                send_sem=send_sem,
                recv_sem=recv_sem,
                device_id=nbr,
                device_id_type=pl.DeviceIdType.MESH,
            )
            rdma.start()
            rdma.wait()
            out_ref[...] = (acc_ref[...] + comm_ref[...]) * (1.0 / m_global)

    return pl.pallas_call(
        body,
        grid=(num_blocks,),
        out_shape=jax.ShapeDtypeStruct((1, n_per), jnp.float32),
        in_specs=[
            pl.BlockSpec(
                (BLOCK_M, n_per), lambda i: (i, 0), memory_space=pltpu.VMEM
            )
        ],
        out_specs=pl.BlockSpec(
            (1, n_per), lambda i: (0, 0), memory_space=pltpu.VMEM
        ),
        scratch_shapes=[
            pltpu.VMEM((1, n_per), jnp.float32),
            pltpu.VMEM((1, n_per), jnp.float32),
            pltpu.SemaphoreType.DMA,
            pltpu.SemaphoreType.DMA,
        ],
        compiler_params=pltpu.CompilerParams(
            collective_id=0,
            dimension_semantics=("arbitrary",),
        ),
    )(x)
